# baseline (speedup 1.0000x reference)
import functools

import jax
import jax.numpy as jnp
from jax import lax
from jax.experimental import pallas as pl
from jax.experimental.pallas import tpu as pltpu
from jax.experimental.pallas import tpu_sc as plsc

NUM_SEGMENTS = 8
_CHUNK = 128


def _pick_block(n, candidates):
    for b in candidates:
        if n % b == 0:
            return b
    return 1


def _sc_segment_sums(x, batch):
    n, c = x.shape
    s = NUM_SEGMENTS
    nfull = n // _CHUNK
    tail = n - nfull * _CHUNK
    nw = 32
    jmax = (nfull + nw - 1) // nw

    mesh = plsc.VectorSubcoreMesh(core_axis_name="c", subcore_axis_name="s")

    @functools.partial(
        pl.kernel, mesh=mesh,
        out_type=jax.ShapeDtypeStruct((2, s, c), jnp.float32),
        scratch_types=[
            pltpu.VMEM((_CHUNK, c), jnp.float32),
            pltpu.VMEM((_CHUNK,), jnp.int32),
            pltpu.VMEM((tail if tail else 8, c), jnp.float32),
            pltpu.VMEM((tail if tail else 8,), jnp.int32),
            pltpu.VMEM((s, c), jnp.float32),
            pltpu.VMEM_SHARED((s, c), jnp.float32),
        ],
    )
    def sums_kernel(x_hbm, seg_hbm, out_hbm, rows_v, idx_v, trows_v, tidx_v,
                    zero_v, acc_sh):
        cid = lax.axis_index("c")
        sid = lax.axis_index("s")
        wid = sid * 2 + cid

        @pl.when(sid == 0)
        def _init():
            for rr in range(s):
                for cc in range(c // 16):
                    zero_v[rr, pl.ds(cc * 16, 16)] = jnp.zeros((16,), jnp.float32)
            pltpu.sync_copy(zero_v, acc_sh)

        plsc.subcore_barrier()

        for j in range(jmax):
            chunk = wid + nw * j

            @pl.when(chunk < nfull)
            def _do():
                base = chunk * _CHUNK
                pltpu.sync_copy(x_hbm.at[pl.ds(base, _CHUNK)], rows_v)
                pltpu.sync_copy(seg_hbm.at[pl.ds(base, _CHUNK)], idx_v)
                pltpu.sync_copy(rows_v, acc_sh.at[idx_v], add=True)

        if tail:
            @pl.when(wid == nw - 1)
            def _tail():
                base = nfull * _CHUNK
                pltpu.sync_copy(x_hbm.at[pl.ds(base, tail)], trows_v)
                pltpu.sync_copy(seg_hbm.at[pl.ds(base, tail)], tidx_v)
                pltpu.sync_copy(trows_v, acc_sh.at[tidx_v], add=True)

        plsc.subcore_barrier()

        @pl.when(sid == 0)
        def _emit():
            pltpu.sync_copy(acc_sh, out_hbm.at[cid])

    return sums_kernel(x, batch)


def _combine_kernel(part, segf, w1, b1, w2, b2, out, *, n):
    sums = part[0] + part[1]
    seg = segf[0, 0, :]
    iota = jax.lax.broadcasted_iota(jnp.int32, (NUM_SEGMENTS, n), 0)
    counts = jnp.sum((iota == seg[None, :]).astype(jnp.float32), axis=1,
                     keepdims=True)
    h = sums / jnp.maximum(counts, 1.0)
    h = jax.lax.dot_general(h, w1[...], (((1,), (1,)), ((), ())),
                            precision=jax.lax.Precision.HIGHEST,
                            preferred_element_type=jnp.float32) + b1[...]
    h = jnp.where(h > 0, h, 0.01 * h)
    h = jax.lax.dot_general(h, w2[...], (((1,), (1,)), ((), ())),
                            precision=jax.lax.Precision.HIGHEST,
                            preferred_element_type=jnp.float32) + b2[...]
    h = jnp.where(h > 0, h, 0.01 * h)
    out[...] = h


def _tile_kernel(h, out):
    out[...] = jnp.broadcast_to(h[...][None, :, :], out.shape)


def kernel(x, batch, W1, b1, W2, b2):
    n, c = x.shape
    s = NUM_SEGMENTS

    partials = _sc_segment_sums(x, batch)

    h = pl.pallas_call(
        functools.partial(_combine_kernel, n=n),
        grid=(1,),
        in_specs=[
            pl.BlockSpec((2, s, c), lambda i: (0, 0, 0)),
            pl.BlockSpec((1, 1, n), lambda i: (0, 0, 0)),
            pl.BlockSpec((c, c), lambda i: (0, 0)),
            pl.BlockSpec((1, c), lambda i: (0, 0)),
            pl.BlockSpec((c, c), lambda i: (0, 0)),
            pl.BlockSpec((1, c), lambda i: (0, 0)),
        ],
        out_specs=pl.BlockSpec((s, c), lambda i: (0, 0)),
        out_shape=jax.ShapeDtypeStruct((s, c), jnp.float32),
    )(partials, batch.reshape(1, 1, n), W1, b1.reshape(1, c),
      W2, b2.reshape(1, c))

    r = _pick_block(n, (6250, 5000, 2500, 1250, 1000, 625, 500, 400, 250, 200,
                        125, 100, 80, 50, 40, 25, 20, 16, 10, 8, 5, 4, 2))
    tiles = n // r
    out = pl.pallas_call(
        _tile_kernel,
        grid=(tiles,),
        in_specs=[pl.BlockSpec((s, c), lambda i: (0, 0))],
        out_specs=pl.BlockSpec((r, s, c), lambda i: (i, 0, 0)),
        out_shape=jax.ShapeDtypeStruct((n, s, c), jnp.float32),
    )(h)
    return out.reshape(n * s, c)


# speedup vs baseline: 1.2518x; 1.2518x over previous
import functools

import jax
import jax.numpy as jnp
from jax import lax
from jax.experimental import pallas as pl
from jax.experimental.pallas import tpu as pltpu
from jax.experimental.pallas import tpu_sc as plsc

NUM_SEGMENTS = 8
_CHUNK = 128
_NW = 32


def _pick_block(n, candidates):
    for b in candidates:
        if n % b == 0:
            return b
    return 1


def _sc_segment_sums(x, batch, m):
    n, c = x.shape
    s = NUM_SEGMENTS
    nrows = n - m
    nfull = nrows // _CHUNK
    tail = nrows - nfull * _CHUNK
    jmax = (nfull + _NW - 1) // _NW

    mesh = plsc.VectorSubcoreMesh(core_axis_name="c", subcore_axis_name="s")

    @functools.partial(
        pl.kernel, mesh=mesh,
        out_type=jax.ShapeDtypeStruct((2, s, c), jnp.float32),
        scratch_types=[
            pltpu.VMEM((2, _CHUNK, c), jnp.float32),
            pltpu.VMEM((2, _CHUNK), jnp.int32),
            pltpu.VMEM((tail if tail else 8, c), jnp.float32),
            pltpu.VMEM((tail if tail else 8,), jnp.int32),
            pltpu.VMEM((s, c), jnp.float32),
            pltpu.VMEM_SHARED((s, c), jnp.float32),
            pltpu.SemaphoreType.DMA,
            pltpu.SemaphoreType.DMA,
            pltpu.SemaphoreType.DMA,
            pltpu.SemaphoreType.DMA,
        ],
    )
    def sums_kernel(x_hbm, seg_hbm, out_hbm, rows2_v, idx2_v, trows_v, tidx_v,
                    zero_v, acc_sh, xs0, xs1, is0, is1):
        cid = lax.axis_index("c")
        sid = lax.axis_index("s")
        wid = sid * 2 + cid
        xsem = (xs0, xs1)
        isem = (is0, is1)

        @pl.when(sid == 0)
        def _init():
            for rr in range(s):
                for cc in range(c // 16):
                    zero_v[rr, pl.ds(cc * 16, 16)] = jnp.zeros((16,), jnp.float32)
            pltpu.sync_copy(zero_v, acc_sh)

        plsc.subcore_barrier()

        for j in range(jmax):
            chunk = wid + _NW * j

            @pl.when(chunk < nfull)
            def _consume(j=j):
                b = j % 2
                base = pl.multiple_of(m + chunk * _CHUNK, 8)
                pltpu.sync_copy(x_hbm.at[pl.ds(base, _CHUNK)], rows2_v.at[b])
                pltpu.sync_copy(seg_hbm.at[pl.ds(base, _CHUNK)], idx2_v.at[b])
                pltpu.sync_copy(rows2_v.at[b], acc_sh.at[idx2_v.at[b]],
                                add=True)

        if tail:
            @pl.when(wid == _NW - 1)
            def _tail():
                base = pl.multiple_of(m + nfull * _CHUNK, 8)
                pltpu.sync_copy(x_hbm.at[pl.ds(base, tail)], trows_v)
                pltpu.sync_copy(seg_hbm.at[pl.ds(base, tail)], tidx_v)
                pltpu.sync_copy(trows_v, acc_sh.at[tidx_v], add=True)

        plsc.subcore_barrier()

        @pl.when(sid == 0)
        def _emit():
            pltpu.sync_copy(acc_sh, out_hbm.at[cid])

    return sums_kernel(x, batch)


def _tc_reduce_kernel(xb, segb, segf, out, sums, *, block, n):
    i = pl.program_id(0)
    k = pl.num_programs(0)

    @pl.when(i == 0)
    def _init():
        sums[...] = jnp.zeros_like(sums)

    seg = segb[0, 0, :]
    iota = jax.lax.broadcasted_iota(jnp.int32, (NUM_SEGMENTS, block), 0)
    onehot = (iota == seg[None, :]).astype(jnp.float32)
    xv = xb[...]
    xh = xv.astype(jnp.bfloat16).astype(jnp.float32)
    xm = (xv - xh).astype(jnp.bfloat16).astype(jnp.float32)
    xl = xv - xh - xm
    acc = jax.lax.dot(onehot, xh, preferred_element_type=jnp.float32)
    acc += jax.lax.dot(onehot, xm, preferred_element_type=jnp.float32)
    acc += jax.lax.dot(onehot, xl, preferred_element_type=jnp.float32)
    sums[...] += acc

    @pl.when(i == k - 1)
    def _finish():
        segall = segf[0, 0, :]
        iota2 = jax.lax.broadcasted_iota(jnp.int32, (NUM_SEGMENTS, n), 0)
        counts = jnp.sum((iota2 == segall[None, :]).astype(jnp.float32),
                         axis=1, keepdims=True)
        out[0] = sums[...]
        out[1] = jnp.broadcast_to(counts, sums.shape)


def _tile_kernel(tc_part, sc_part, w1, b1, w2, b2, out, h_s):
    @pl.when(pl.program_id(0) == 0)
    def _mlp():
        sums = tc_part[0] + sc_part[0] + sc_part[1]
        counts = tc_part[1]
        h = sums / jnp.maximum(counts, 1.0)
        h = jax.lax.dot_general(h, w1[...], (((1,), (1,)), ((), ())),
                                precision=jax.lax.Precision.HIGHEST,
                                preferred_element_type=jnp.float32) + b1[...]
        h = jnp.where(h > 0, h, 0.01 * h)
        h = jax.lax.dot_general(h, w2[...], (((1,), (1,)), ((), ())),
                                precision=jax.lax.Precision.HIGHEST,
                                preferred_element_type=jnp.float32) + b2[...]
        h = jnp.where(h > 0, h, 0.01 * h)
        h_s[...] = h

    out[...] = jnp.broadcast_to(h_s[...][None, :, :], out.shape)


def kernel(x, batch, W1, b1, W2, b2):
    n, c = x.shape
    s = NUM_SEGMENTS

    m = (n * 69 // 100) // (4 * _CHUNK) * (4 * _CHUNK)
    tc_k = 4
    block = m // tc_k

    sc_part = _sc_segment_sums(x, batch, m)

    tc_part = pl.pallas_call(
        functools.partial(_tc_reduce_kernel, block=block, n=n),
        grid=(tc_k,),
        in_specs=[
            pl.BlockSpec((block, c), lambda i: (i, 0)),
            pl.BlockSpec((1, 1, block), lambda i: (i, 0, 0)),
            pl.BlockSpec((1, 1, n), lambda i: (0, 0, 0)),
        ],
        out_specs=pl.BlockSpec((2, s, c), lambda i: (0, 0, 0)),
        out_shape=jax.ShapeDtypeStruct((2, s, c), jnp.float32),
        scratch_shapes=[pltpu.VMEM((s, c), jnp.float32)],
    )(x, batch[:m].reshape(tc_k, 1, block), batch.reshape(1, 1, n))

    r = _pick_block(n, (6250, 5000, 2500, 1250, 1000, 625, 500, 400, 250, 200,
                        125, 100, 80, 50, 40, 25, 20, 16, 10, 8, 5, 4, 2))
    tiles = n // r
    out = pl.pallas_call(
        _tile_kernel,
        grid=(tiles,),
        in_specs=[
            pl.BlockSpec((2, s, c), lambda i: (0, 0, 0)),
            pl.BlockSpec((2, s, c), lambda i: (0, 0, 0)),
            pl.BlockSpec((c, c), lambda i: (0, 0)),
            pl.BlockSpec((1, c), lambda i: (0, 0)),
            pl.BlockSpec((c, c), lambda i: (0, 0)),
            pl.BlockSpec((1, c), lambda i: (0, 0)),
        ],
        out_specs=pl.BlockSpec((r, s, c), lambda i: (i, 0, 0)),
        out_shape=jax.ShapeDtypeStruct((n, s, c), jnp.float32),
        scratch_shapes=[pltpu.VMEM((s, c), jnp.float32)],
    )(tc_part, sc_part, W1, b1.reshape(1, c), W2, b2.reshape(1, c))
    return out.reshape(n * s, c)


# speedup vs baseline: 1.2800x; 1.0225x over previous
import functools

import jax
import jax.numpy as jnp
from jax import lax
from jax.experimental import pallas as pl
from jax.experimental.pallas import tpu as pltpu
from jax.experimental.pallas import tpu_sc as plsc

NUM_SEGMENTS = 8
_CHUNK = 128
_NW = 32


def _pick_block(n, candidates):
    for b in candidates:
        if n % b == 0:
            return b
    return 1


def _sc_segment_sums(x, batch, m):
    n, c = x.shape
    s = NUM_SEGMENTS
    nrows = n - m
    nfull = nrows // _CHUNK
    tail = nrows - nfull * _CHUNK
    jmax = (nfull + _NW - 1) // _NW

    mesh = plsc.VectorSubcoreMesh(core_axis_name="c", subcore_axis_name="s")

    @functools.partial(
        pl.kernel, mesh=mesh,
        out_type=jax.ShapeDtypeStruct((2, s, c), jnp.float32),
        scratch_types=[
            pltpu.VMEM((2, _CHUNK, c), jnp.float32),
            pltpu.VMEM((2, _CHUNK), jnp.int32),
            pltpu.VMEM((tail if tail else 8, c), jnp.float32),
            pltpu.VMEM((tail if tail else 8,), jnp.int32),
            pltpu.VMEM((s, c), jnp.float32),
            pltpu.VMEM_SHARED((s, c), jnp.float32),
            pltpu.SemaphoreType.DMA,
            pltpu.SemaphoreType.DMA,
            pltpu.SemaphoreType.DMA,
            pltpu.SemaphoreType.DMA,
        ],
    )
    def sums_kernel(x_hbm, seg_hbm, out_hbm, rows2_v, idx2_v, trows_v, tidx_v,
                    zero_v, acc_sh, xs0, xs1, is0, is1):
        cid = lax.axis_index("c")
        sid = lax.axis_index("s")
        wid = sid * 2 + cid
        xsem = (xs0, xs1)
        isem = (is0, is1)

        @pl.when(sid == 0)
        def _init():
            for rr in range(s):
                for cc in range(c // 16):
                    zero_v[rr, pl.ds(cc * 16, 16)] = jnp.zeros((16,), jnp.float32)
            pltpu.sync_copy(zero_v, acc_sh)

        plsc.subcore_barrier()

        for j in range(jmax):
            chunk = wid + _NW * j

            @pl.when(chunk < nfull)
            def _consume(j=j):
                b = j % 2
                base = pl.multiple_of(m + chunk * _CHUNK, 8)
                pltpu.sync_copy(x_hbm.at[pl.ds(base, _CHUNK)], rows2_v.at[b])
                pltpu.sync_copy(seg_hbm.at[pl.ds(base, _CHUNK)], idx2_v.at[b])
                pltpu.sync_copy(rows2_v.at[b], acc_sh.at[idx2_v.at[b]],
                                add=True)

        if tail:
            @pl.when(wid == _NW - 1)
            def _tail():
                base = pl.multiple_of(m + nfull * _CHUNK, 8)
                pltpu.sync_copy(x_hbm.at[pl.ds(base, tail)], trows_v)
                pltpu.sync_copy(seg_hbm.at[pl.ds(base, tail)], tidx_v)
                pltpu.sync_copy(trows_v, acc_sh.at[tidx_v], add=True)

        plsc.subcore_barrier()

        @pl.when(sid == 0)
        def _emit():
            pltpu.sync_copy(acc_sh, out_hbm.at[cid])

    return sums_kernel(x, batch)


def _tc_reduce_kernel(xb, segb, segf, out, sums, *, block, n):
    i = pl.program_id(0)
    k = pl.num_programs(0)

    @pl.when(i == 0)
    def _init():
        sums[...] = jnp.zeros_like(sums)

    seg = segb[0, 0, :]
    iota = jax.lax.broadcasted_iota(jnp.int32, (NUM_SEGMENTS, block), 0)
    onehot = (iota == seg[None, :]).astype(jnp.float32)
    xv = xb[...]
    xh = xv.astype(jnp.bfloat16).astype(jnp.float32)
    xm = (xv - xh).astype(jnp.bfloat16).astype(jnp.float32)
    xl = xv - xh - xm
    acc = jax.lax.dot(onehot, xh, preferred_element_type=jnp.float32)
    acc += jax.lax.dot(onehot, xm, preferred_element_type=jnp.float32)
    acc += jax.lax.dot(onehot, xl, preferred_element_type=jnp.float32)
    sums[...] += acc

    @pl.when(i == k - 1)
    def _finish():
        segall = segf[0, 0, :]
        iota2 = jax.lax.broadcasted_iota(jnp.int32, (NUM_SEGMENTS, n), 0)
        counts = jnp.sum((iota2 == segall[None, :]).astype(jnp.float32),
                         axis=1, keepdims=True)
        out[0] = sums[...]
        out[1] = jnp.broadcast_to(counts, sums.shape)


def _tile_kernel(tc_part, sc_part, w1, b1, w2, b2, out, h_s):
    @pl.when(pl.program_id(0) == 0)
    def _mlp():
        sums = tc_part[0] + sc_part[0] + sc_part[1]
        counts = tc_part[1]
        h = sums / jnp.maximum(counts, 1.0)
        h = jax.lax.dot_general(h, w1[...], (((1,), (1,)), ((), ())),
                                precision=jax.lax.Precision.HIGHEST,
                                preferred_element_type=jnp.float32) + b1[...]
        h = jnp.where(h > 0, h, 0.01 * h)
        h = jax.lax.dot_general(h, w2[...], (((1,), (1,)), ((), ())),
                                precision=jax.lax.Precision.HIGHEST,
                                preferred_element_type=jnp.float32) + b2[...]
        h = jnp.where(h > 0, h, 0.01 * h)
        h_s[...] = h

    out[...] = jnp.broadcast_to(h_s[...][None, :, :], out.shape)


def kernel(x, batch, W1, b1, W2, b2):
    n, c = x.shape
    s = NUM_SEGMENTS

    m = (n * 85 // 100) // (4 * _CHUNK) * (4 * _CHUNK)
    tc_k = 4
    block = m // tc_k

    sc_part = _sc_segment_sums(x, batch, m)

    tc_part = pl.pallas_call(
        functools.partial(_tc_reduce_kernel, block=block, n=n),
        grid=(tc_k,),
        in_specs=[
            pl.BlockSpec((block, c), lambda i: (i, 0)),
            pl.BlockSpec((1, 1, block), lambda i: (i, 0, 0)),
            pl.BlockSpec((1, 1, n), lambda i: (0, 0, 0)),
        ],
        out_specs=pl.BlockSpec((2, s, c), lambda i: (0, 0, 0)),
        out_shape=jax.ShapeDtypeStruct((2, s, c), jnp.float32),
        scratch_shapes=[pltpu.VMEM((s, c), jnp.float32)],
    )(x, batch[:m].reshape(tc_k, 1, block), batch.reshape(1, 1, n))

    r = _pick_block(n, (6250, 5000, 2500, 1250, 1000, 625, 500, 400, 250, 200,
                        125, 100, 80, 50, 40, 25, 20, 16, 10, 8, 5, 4, 2))
    tiles = n // r
    out = pl.pallas_call(
        _tile_kernel,
        grid=(tiles,),
        in_specs=[
            pl.BlockSpec((2, s, c), lambda i: (0, 0, 0)),
            pl.BlockSpec((2, s, c), lambda i: (0, 0, 0)),
            pl.BlockSpec((c, c), lambda i: (0, 0)),
            pl.BlockSpec((1, c), lambda i: (0, 0)),
            pl.BlockSpec((c, c), lambda i: (0, 0)),
            pl.BlockSpec((1, c), lambda i: (0, 0)),
        ],
        out_specs=pl.BlockSpec((r, s, c), lambda i: (i, 0, 0)),
        out_shape=jax.ShapeDtypeStruct((n, s, c), jnp.float32),
        scratch_shapes=[pltpu.VMEM((s, c), jnp.float32)],
    )(tc_part, sc_part, W1, b1.reshape(1, c), W2, b2.reshape(1, c))
    return out.reshape(n * s, c)


# speedup vs baseline: 1.5404x; 1.2035x over previous
import functools

import jax
import jax.numpy as jnp
from jax.experimental import pallas as pl
from jax.experimental.pallas import tpu as pltpu

NUM_SEGMENTS = 8


def _pick_block(n, candidates):
    for b in candidates:
        if n % b == 0:
            return b
    return 1


def _fused_kernel(xb, segb, w1, b1, w2, b2, out, sums, counts, h_s, *,
                  block, k):
    i = pl.program_id(0)

    @pl.when(i == 0)
    def _init():
        sums[...] = jnp.zeros_like(sums)
        counts[...] = jnp.zeros_like(counts)

    @pl.when(i < k)
    def _reduce():
        seg = segb[0, 0, :]
        iota = jax.lax.broadcasted_iota(jnp.int32, (NUM_SEGMENTS, block), 0)
        onehot = (iota == seg[None, :]).astype(jnp.float32)
        xv = xb[...]
        xh = xv.astype(jnp.bfloat16).astype(jnp.float32)
        xm = (xv - xh).astype(jnp.bfloat16).astype(jnp.float32)
        xl = xv - xh - xm
        acc = jax.lax.dot(onehot, xh, preferred_element_type=jnp.float32)
        acc += jax.lax.dot(onehot, xm, preferred_element_type=jnp.float32)
        acc += jax.lax.dot(onehot, xl, preferred_element_type=jnp.float32)
        sums[...] += acc
        counts[...] += jnp.sum(onehot, axis=1, keepdims=True)

    @pl.when(i == k - 1)
    def _mlp():
        h = sums[...] / jnp.maximum(counts[...], 1.0)
        h = jax.lax.dot_general(h, w1[...], (((1,), (1,)), ((), ())),
                                precision=jax.lax.Precision.HIGHEST,
                                preferred_element_type=jnp.float32) + b1[...]
        h = jnp.where(h > 0, h, 0.01 * h)
        h = jax.lax.dot_general(h, w2[...], (((1,), (1,)), ((), ())),
                                precision=jax.lax.Precision.HIGHEST,
                                preferred_element_type=jnp.float32) + b2[...]
        h = jnp.where(h > 0, h, 0.01 * h)
        h_s[...] = h

    @pl.when(i >= k)
    def _tile():
        out[...] = jnp.broadcast_to(h_s[...][None, :, :], out.shape)


def kernel(x, batch, W1, b1, W2, b2):
    n, c = x.shape
    s = NUM_SEGMENTS

    block = _pick_block(n, (5000, 2000, 1000, 400, 200, 80, 40, 16, 8))
    k = n // block
    seg3 = batch.reshape(k, 1, block)

    r = _pick_block(n, (5000, 2500, 1250, 1000, 625, 500, 400, 250, 200,
                        125, 100, 80, 50, 40, 25, 20, 16, 10, 8, 5, 4, 2))
    tiles = n // r

    out = pl.pallas_call(
        functools.partial(_fused_kernel, block=block, k=k),
        grid=(k + tiles,),
        in_specs=[
            pl.BlockSpec((block, c), lambda i: (jnp.minimum(i, k - 1), 0)),
            pl.BlockSpec((1, 1, block), lambda i: (jnp.minimum(i, k - 1), 0, 0)),
            pl.BlockSpec((c, c), lambda i: (0, 0)),
            pl.BlockSpec((1, c), lambda i: (0, 0)),
            pl.BlockSpec((c, c), lambda i: (0, 0)),
            pl.BlockSpec((1, c), lambda i: (0, 0)),
        ],
        out_specs=pl.BlockSpec(
            (r, s, c), lambda i: (jnp.maximum(i - k, 0), 0, 0)),
        out_shape=jax.ShapeDtypeStruct((n, s, c), jnp.float32),
        scratch_shapes=[
            pltpu.VMEM((s, c), jnp.float32),
            pltpu.VMEM((s, 1), jnp.float32),
            pltpu.VMEM((s, c), jnp.float32),
        ],
    )(x, seg3, W1, b1.reshape(1, c), W2, b2.reshape(1, c))
    return out.reshape(n * s, c)
